# baseline (speedup 1.0000x reference)
import functools

import jax
import jax.numpy as jnp
from jax.experimental import pallas as pl
from jax.experimental.pallas import tpu as pltpu
from jax.experimental.pallas import tpu_sc as plsc

_B = 256
_F = 128
_W = 16


def _make_sc_meta(n, R, grid):
    mesh = plsc.VectorSubcoreMesh(core_axis_name="c", subcore_axis_name="s")
    gp = ((grid + 15) // 16) * 16
    nbits = max(1, (n - 1).bit_length())

    @functools.partial(
        pl.kernel, mesh=mesh,
        out_type=[jax.ShapeDtypeStruct((2, _B), jnp.float32),
                  jax.ShapeDtypeStruct((2, gp), jnp.int32),
                  jax.ShapeDtypeStruct((2, gp), jnp.int32)],
        scratch_types=[pltpu.VMEM((16,), jnp.int32),
                       pltpu.VMEM((16,), jnp.int32),
                       pltpu.VMEM((16,), jnp.int32),
                       pltpu.VMEM((16,), jnp.int32),
                       pltpu.VMEM((16,), jnp.float32),
                       pltpu.VMEM((gp,), jnp.int32),
                       pltpu.VMEM((gp,), jnp.int32),
                       pltpu.SemaphoreType.DMA],
    )
    def sc_meta(b1_hbm, b2_hbm, cnt_hbm, base_hbm, mx_hbm,
                ia, ib, va, vb, cv, bsv, mxv, dsem):
        cid = jax.lax.axis_index("c")
        sid = jax.lax.axis_index("s")
        lanes = jax.lax.iota(jnp.int32, 16)

        def gather2(bh, idx_a, idx_b):
            ia[...] = idx_a
            ib[...] = idx_b
            ca = pltpu.async_copy(bh.at[ia], va, dsem)
            cb = pltpu.async_copy(bh.at[ib], vb, dsem)
            ca.wait()
            cb.wait()
            return va[...], vb[...]

        def run(bh):
            tgt_a = lanes + 16 * sid
            tgt_b = tgt_a + 1
            lo_a = jnp.zeros((16,), jnp.int32)
            hi_a = jnp.full((16,), n, jnp.int32)
            lo_b = lo_a
            hi_b = hi_a
            for _ in range(nbits):
                mid_a = (lo_a + hi_a) >> 1
                mid_b = (lo_b + hi_b) >> 1
                xa, xb = gather2(bh, mid_a, mid_b)
                pa = xa < tgt_a
                pb = xb < tgt_b
                lo_a = jnp.where(pa, mid_a + 1, lo_a)
                hi_a = jnp.where(pa, hi_a, mid_a)
                lo_b = jnp.where(pb, mid_b + 1, lo_b)
                hi_b = jnp.where(pb, hi_b, mid_b)
            cv[...] = (lo_b - lo_a).astype(jnp.float32)
            pltpu.sync_copy(cv, cnt_hbm.at[cid, pl.ds(16 * sid, 16)])

            @pl.when(sid == 0)
            def _():
                for k in range(gp // 16):
                    t = jnp.minimum(lanes + 16 * k, grid - 1)
                    xa, xb = gather2(bh, t * R, t * R + (R - 1))
                    bsv[pl.ds(16 * k, 16)] = xa
                    mxv[pl.ds(16 * k, 16)] = xb
                pltpu.sync_copy(bsv, base_hbm.at[cid])
                pltpu.sync_copy(mxv, mx_hbm.at[cid])

        @pl.when(cid == 0)
        def _():
            run(b1_hbm)

        @pl.when(cid == 1)
        def _():
            run(b2_hbm)

    return sc_meta


def _mm(a, b):
    return jax.lax.dot_general(a, b, (((1,), (0,)), ((), ())),
                               preferred_element_type=jnp.float32)


def _relu(x):
    return jnp.maximum(x, 0.0)


def _make_pass1(R, grid):
    def body(base, mx, br1, br2, x1, x2, u1, u2,
             A1, Bu1, b11, A2, Bu2, b12,
             s1o, s2o, t1, t2, sa1, sa2):
        step = pl.program_id(0)

        @pl.when(step == 0)
        def _init():
            t1[...] = jnp.zeros((_B + _W, _F), jnp.float32)
            t2[...] = jnp.zeros((_B + _W, _F), jnp.float32)
            t1[:_B, :] = _mm(u1[...], Bu1[...]) + b11[...]
            t2[:_B, :] = _mm(u2[...], Bu2[...]) + b12[...]
            sa1[...] = jnp.zeros((_B + _W, _F), jnp.float32)
            sa2[...] = jnp.zeros((_B + _W, _F), jnp.float32)

        iw_s = jax.lax.broadcasted_iota(jnp.int32, (_W, R), 0)
        if_s = jax.lax.broadcasted_iota(jnp.int32, (_B, R), 0)
        for g, (br, x, a, t, sa) in enumerate((
                (br1, x1, A1, t1, sa1),
                (br2, x2, A2, t2, sa2))):
            b0 = base[g, step]
            narrow = (mx[g, step] - b0) < _W

            @pl.when(narrow)
            def _fast():
                oht = (br[0] == iw_s + b0).astype(jnp.float32)
                oh = oht.T
                h = _relu(_mm(x[...], a[...])
                          + _mm(oh, t[pl.ds(b0, _W), :]))
                sa[pl.ds(b0, _W), :] += _mm(oht, h)

            @pl.when(jnp.logical_not(narrow))
            def _slow():
                oht = (br[0] == if_s).astype(jnp.float32)
                oh = oht.T
                h = _relu(_mm(x[...], a[...]) + _mm(oh, t[:_B, :]))
                sa[:_B, :] += _mm(oht, h)

        @pl.when(step == grid - 1)
        def _fini():
            s1o[...] = sa1[:_B, :]
            s2o[...] = sa2[:_B, :]
    return body


def _make_pass2(R, grid):
    def body(base, mx, br1, br2, x1, x2, u1, u2, s11, s12,
             cn1, cn2,
             A1, Bu1, b11, W21, b21, Va1, Vu1, d11, V21, d21,
             A2, Bu2, b12, W22, b22, Va2, Vu2, d12, V22, d22,
             Wfa, Wfb, bf1, Wf2, bf2,
             out,
             tc11, tc21, G1, un1, a1, tc12, tc22, G2, un2, a2):
        step = pl.program_id(0)

        @pl.when(step == 0)
        def _init():
            for (s1, cn, u, a, bu, b1, w2, b2, va, vu, d1, v2, d2,
                 tc1, tc2, g, un, acc) in (
                    (s11, cn1, u1, A1, Bu1, b11, W21, b21, Va1, Vu1, d11,
                     V21, d21, tc11, tc21, G1, un1, a1),
                    (s12, cn2, u2, A2, Bu2, b12, W22, b22, Va2, Vu2, d12,
                     V22, d22, tc12, tc22, G2, un2, a2)):
                agg1 = _mm(s1[...], w2[...]) + cn[...] * b2[...]
                zz = _relu(_mm(agg1, va[...]) + _mm(u[...], vu[...]) + d1[...])
                unew = _mm(zz, v2[...]) + d2[...]
                un[...] = unew
                tc1[...] = jnp.zeros((_B + _W, _F), jnp.float32)
                tc2[...] = jnp.zeros((_B + _W, _F), jnp.float32)
                tc1[:_B, :] = _mm(u[...], bu[...]) + b1[...]
                g[...] = _mm(w2[...], a[...])
                e = _mm(b2[...], a[...])
                tc2[:_B, :] = _mm(unew, bu[...]) + b1[...] + e
                acc[...] = jnp.zeros((_B + _W, _F), jnp.float32)
            zf = _relu(_mm(un1[...], Wfa[...]) + _mm(un2[...], Wfb[...])
                       + bf1[...])
            out[0, :, :] = _mm(zf, Wf2[...]) + bf2[...]

        iw_s = jax.lax.broadcasted_iota(jnp.int32, (_W, R), 0)
        if_s = jax.lax.broadcasted_iota(jnp.int32, (_B, R), 0)
        for g, (br, x, a, tc1, tc2, gm, acc) in enumerate((
                (br1, x1, A1, tc11, tc21, G1, a1),
                (br2, x2, A2, tc12, tc22, G2, a2))):
            b0 = base[g, step]
            narrow = (mx[g, step] - b0) < _W

            @pl.when(narrow)
            def _fast():
                oht = (br[0] == iw_s + b0).astype(jnp.float32)
                oh = oht.T
                h = _relu(_mm(x[...], a[...])
                          + _mm(oh, tc1[pl.ds(b0, _W), :]))
                h2 = _relu(_mm(h, gm[...])
                           + _mm(oh, tc2[pl.ds(b0, _W), :]))
                acc[pl.ds(b0, _W), :] += _mm(oht, h2)

            @pl.when(jnp.logical_not(narrow))
            def _slow():
                oht = (br[0] == if_s).astype(jnp.float32)
                oh = oht.T
                h = _relu(_mm(x[...], a[...]) + _mm(oh, tc1[:_B, :]))
                h2 = _relu(_mm(h, gm[...]) + _mm(oh, tc2[:_B, :]))
                acc[:_B, :] += _mm(oht, h2)

        @pl.when(step == grid - 1)
        def _fini():
            u2s = []
            for cn, w2, b2, va, vu, d1, v2, d2, un, acc in (
                    (cn1, W21, b21, Va1, Vu1, d11, V21, d21, un1, a1),
                    (cn2, W22, b22, Va2, Vu2, d12, V22, d22, un2, a2)):
                agg2 = _mm(acc[:_B, :], w2[...]) + cn[...] * b2[...]
                zz = _relu(_mm(agg2, va[...]) + _mm(un[...], vu[...])
                           + d1[...])
                u2s.append(_mm(zz, v2[...]) + d2[...])
            zf = _relu(_mm(u2s[0], Wfa[...]) + _mm(u2s[1], Wfb[...])
                       + bf1[...])
            out[1, :, :] = _mm(zf, Wf2[...]) + bf2[...]
    return body


def kernel(x1, u1, x2, u2, batch1, batch2, params):
    n = x1.shape[0]
    R = 5000 if n % 5000 == 0 else 8
    assert n % R == 0
    grid = n // R

    (W11, b11), (W21, b21) = params['gnn1_node']
    (W12, b12), (W22, b22) = params['gnn2_node']
    (Vg11, d11), (Vg21, d21) = params['gnn1_glob']
    (Vg12, d12), (Vg22, d22) = params['gnn2_glob']
    (Wf1, bf1), (Wf2, bf2) = params['final']

    A1, Bu1 = W11[:_F], W11[_F:]
    A2, Bu2 = W12[:_F], W12[_F:]
    Va1, Vu1 = Vg11[:_F], Vg11[_F:]
    Va2, Vu2 = Vg12[:_F], Vg12[_F:]
    Wfa, Wfb = Wf1[:_F], Wf1[_F:]

    row = lambda v: v.reshape(1, -1)
    br1 = batch1.reshape(grid, 1, R)
    br2 = batch2.reshape(grid, 1, R)

    sm_spec = pl.BlockSpec(memory_space=pltpu.SMEM)
    br_spec = pl.BlockSpec((1, 1, R), lambda i: (i, 0, 0))
    x_spec = pl.BlockSpec((R, _F), lambda i: (i, 0))
    full = lambda arr: pl.BlockSpec(arr.shape, lambda i: (0,) * arr.ndim)
    acc_spec = pl.BlockSpec((_B, _F), lambda i: (0, 0))
    cnt_spec = pl.BlockSpec((_B, 1), lambda i: (0, 0))
    f32 = jnp.float32
    vmem = lambda shape: pltpu.VMEM(shape, f32)

    cnt, base_p, mx_p = _make_sc_meta(n, R, grid)(batch1, batch2)
    base = base_p[:, :grid]
    mx = mx_p[:, :grid]
    cn1 = cnt[0].reshape(_B, 1)
    cn2 = cnt[1].reshape(_B, 1)

    p1_weights = (A1, Bu1, row(b11), A2, Bu2, row(b12))
    s11, s12 = pl.pallas_call(
        _make_pass1(R, grid),
        grid=(grid,),
        in_specs=[sm_spec, sm_spec, br_spec, br_spec,
                  x_spec, x_spec, full(u1), full(u2)]
                 + [full(w) for w in p1_weights],
        out_specs=[acc_spec, acc_spec],
        out_shape=[jax.ShapeDtypeStruct((_B, _F), f32),
                   jax.ShapeDtypeStruct((_B, _F), f32)],
        scratch_shapes=[vmem((_B + _W, _F)), vmem((_B + _W, _F)),
                        vmem((_B + _W, _F)), vmem((_B + _W, _F))],
        compiler_params=pltpu.CompilerParams(
            dimension_semantics=("arbitrary",)),
    )(base, mx, br1, br2, x1, x2, u1, u2, *p1_weights)

    p2_weights = (A1, Bu1, row(b11), W21, row(b21), Va1, Vu1, row(d11),
                  Vg21, row(d21),
                  A2, Bu2, row(b12), W22, row(b22), Va2, Vu2, row(d12),
                  Vg22, row(d22),
                  Wfa, Wfb, row(bf1), Wf2, row(bf2))
    out = pl.pallas_call(
        _make_pass2(R, grid),
        grid=(grid,),
        in_specs=[sm_spec, sm_spec, br_spec, br_spec,
                  x_spec, x_spec, full(u1), full(u2), acc_spec, acc_spec,
                  cnt_spec, cnt_spec] + [full(w) for w in p2_weights],
        out_specs=pl.BlockSpec((2, _B, 2), lambda i: (0, 0, 0)),
        out_shape=jax.ShapeDtypeStruct((2, _B, 2), f32),
        scratch_shapes=[vmem((_B + _W, _F)), vmem((_B + _W, _F)),
                        vmem((_F, _F)), vmem((_B, _F)),
                        vmem((_B + _W, _F)), vmem((_B + _W, _F)),
                        vmem((_B + _W, _F)), vmem((_F, _F)),
                        vmem((_B, _F)), vmem((_B + _W, _F))],
        compiler_params=pltpu.CompilerParams(
            dimension_semantics=("arbitrary",)),
    )(base, mx, br1, br2, x1, x2, u1, u2, s11, s12, cn1, cn2,
      *p2_weights)
    return out


# speedup vs baseline: 1.3731x; 1.3731x over previous
import functools

import jax
import jax.numpy as jnp
from jax.experimental import pallas as pl
from jax.experimental.pallas import tpu as pltpu
from jax.experimental.pallas import tpu_sc as plsc

_B = 256
_F = 128
_W = 16


def _make_sc_meta(n, R, grid):
    mesh = plsc.VectorSubcoreMesh(core_axis_name="c", subcore_axis_name="s")
    gp = ((grid + 15) // 16) * 16
    nbits = max(1, (n - 1).bit_length())

    @functools.partial(
        pl.kernel, mesh=mesh,
        out_type=[jax.ShapeDtypeStruct((2, _B), jnp.float32),
                  jax.ShapeDtypeStruct((2, gp), jnp.int32),
                  jax.ShapeDtypeStruct((2, gp), jnp.int32)],
        scratch_types=[pltpu.VMEM((16,), jnp.int32),
                       pltpu.VMEM((16,), jnp.int32),
                       pltpu.VMEM((16,), jnp.int32),
                       pltpu.VMEM((16,), jnp.int32),
                       pltpu.VMEM((16,), jnp.float32),
                       pltpu.VMEM((gp,), jnp.int32),
                       pltpu.VMEM((gp,), jnp.int32),
                       pltpu.SemaphoreType.DMA],
    )
    def sc_meta(b1_hbm, b2_hbm, cnt_hbm, base_hbm, mx_hbm,
                ia, ib, va, vb, cv, bsv, mxv, dsem):
        cid = jax.lax.axis_index("c")
        sid = jax.lax.axis_index("s")
        lanes = jax.lax.iota(jnp.int32, 16)

        def gather2(bh, idx_a, idx_b):
            ia[...] = idx_a
            ib[...] = idx_b
            ca = pltpu.async_copy(bh.at[ia], va, dsem)
            cb = pltpu.async_copy(bh.at[ib], vb, dsem)
            ca.wait()
            cb.wait()
            return va[...], vb[...]

        def run(bh):
            tgt_a = lanes + 16 * sid
            tgt_b = tgt_a + 1
            lo_a = jnp.zeros((16,), jnp.int32)
            hi_a = jnp.full((16,), n, jnp.int32)
            lo_b = lo_a
            hi_b = hi_a
            for _ in range(nbits):
                mid_a = (lo_a + hi_a) >> 1
                mid_b = (lo_b + hi_b) >> 1
                xa, xb = gather2(bh, mid_a, mid_b)
                pa = xa < tgt_a
                pb = xb < tgt_b
                lo_a = jnp.where(pa, mid_a + 1, lo_a)
                hi_a = jnp.where(pa, hi_a, mid_a)
                lo_b = jnp.where(pb, mid_b + 1, lo_b)
                hi_b = jnp.where(pb, hi_b, mid_b)
            cv[...] = (lo_b - lo_a).astype(jnp.float32)
            pltpu.sync_copy(cv, cnt_hbm.at[cid, pl.ds(16 * sid, 16)])

            @pl.when(sid == 0)
            def _():
                for k in range(gp // 16):
                    t = jnp.minimum(lanes + 16 * k, grid - 1)
                    xa, xb = gather2(bh, t * R, t * R + (R - 1))
                    bsv[pl.ds(16 * k, 16)] = xa
                    mxv[pl.ds(16 * k, 16)] = xb
                pltpu.sync_copy(bsv, base_hbm.at[cid])
                pltpu.sync_copy(mxv, mx_hbm.at[cid])

        @pl.when(cid == 0)
        def _():
            run(b1_hbm)

        @pl.when(cid == 1)
        def _():
            run(b2_hbm)

    return sc_meta


def _mm(a, b):
    return jax.lax.dot_general(a, b, (((1,), (0,)), ((), ())),
                               preferred_element_type=jnp.float32)


def _relu(x):
    return jnp.maximum(x, 0.0)


def _make_pass1(R, grid):
    def body(base, mx, br1, br2, x1, x2, u1, u2,
             A1, Bu1, b11, A2, Bu2, b12,
             s1o, s2o, t1, t2, sa1, sa2):
        step = pl.program_id(0)

        @pl.when(step == 0)
        def _init():
            t1[...] = jnp.zeros((_B + _W, _F), jnp.float32)
            t2[...] = jnp.zeros((_B + _W, _F), jnp.float32)
            t1[:_B, :] = _mm(u1[...], Bu1[...]) + b11[...]
            t2[:_B, :] = _mm(u2[...], Bu2[...]) + b12[...]
            sa1[...] = jnp.zeros((_B + _W, _F), jnp.float32)
            sa2[...] = jnp.zeros((_B + _W, _F), jnp.float32)

        iw_s = jax.lax.broadcasted_iota(jnp.int32, (_W, R), 0)
        if_s = jax.lax.broadcasted_iota(jnp.int32, (_B, R), 0)
        for g, (br, x, a, t, sa) in enumerate((
                (br1, x1, A1, t1, sa1),
                (br2, x2, A2, t2, sa2))):
            b0 = base[g, step]
            narrow = (mx[g, step] - b0) < _W

            @pl.when(narrow)
            def _fast():
                oht = (br[0] == iw_s + b0).astype(jnp.float32)
                oh = oht.T
                h = _relu(_mm(x[...], a[...])
                          + _mm(oh, t[pl.ds(b0, _W), :]))
                sa[pl.ds(b0, _W), :] += _mm(oht, h)

            @pl.when(jnp.logical_not(narrow))
            def _slow():
                oht = (br[0] == if_s).astype(jnp.float32)
                oh = oht.T
                h = _relu(_mm(x[...], a[...]) + _mm(oh, t[:_B, :]))
                sa[:_B, :] += _mm(oht, h)

        @pl.when(step == grid - 1)
        def _fini():
            s1o[...] = sa1[:_B, :]
            s2o[...] = sa2[:_B, :]
    return body


def _make_pass2(R, grid):
    def body(base, mx, br1, br2, x1, x2, u1, u2, s11, s12,
             cn1, cn2,
             A1, Bu1, b11, W21, b21, Va1, Vu1, d11, V21, d21,
             A2, Bu2, b12, W22, b22, Va2, Vu2, d12, V22, d22,
             Wfa, Wfb, bf1, Wf2, bf2,
             out,
             tc11, tc21, G1, un1, a1, tc12, tc22, G2, un2, a2):
        step = pl.program_id(0)

        @pl.when(step == 0)
        def _init():
            for (s1, cn, u, a, bu, b1, w2, b2, va, vu, d1, v2, d2,
                 tc1, tc2, g, un, acc) in (
                    (s11, cn1, u1, A1, Bu1, b11, W21, b21, Va1, Vu1, d11,
                     V21, d21, tc11, tc21, G1, un1, a1),
                    (s12, cn2, u2, A2, Bu2, b12, W22, b22, Va2, Vu2, d12,
                     V22, d22, tc12, tc22, G2, un2, a2)):
                agg1 = _mm(s1[...], w2[...]) + cn[...] * b2[...]
                zz = _relu(_mm(agg1, va[...]) + _mm(u[...], vu[...]) + d1[...])
                unew = _mm(zz, v2[...]) + d2[...]
                un[...] = unew
                tc1[...] = jnp.zeros((_B + _W, _F), jnp.float32)
                tc2[...] = jnp.zeros((_B + _W, _F), jnp.float32)
                tc1[:_B, :] = _mm(u[...], bu[...]) + b1[...]
                g[...] = _mm(w2[...], a[...])
                e = _mm(b2[...], a[...])
                tc2[:_B, :] = _mm(unew, bu[...]) + b1[...] + e
                acc[...] = jnp.zeros((_B + _W, _F), jnp.float32)
            zf = _relu(_mm(un1[...], Wfa[...]) + _mm(un2[...], Wfb[...])
                       + bf1[...])
            out[0, :, :] = _mm(zf, Wf2[...]) + bf2[...]

        iw_s = jax.lax.broadcasted_iota(jnp.int32, (_W, R), 0)
        if_s = jax.lax.broadcasted_iota(jnp.int32, (_B, R), 0)
        for g, (br, x, a, tc1, tc2, gm, acc) in enumerate((
                (br1, x1, A1, tc11, tc21, G1, a1),
                (br2, x2, A2, tc12, tc22, G2, a2))):
            b0 = base[g, step]
            narrow = (mx[g, step] - b0) < _W

            @pl.when(narrow)
            def _fast():
                oht = (br[0] == iw_s + b0).astype(jnp.float32)
                oh = oht.T
                h = _relu(_mm(x[...], a[...])
                          + _mm(oh, tc1[pl.ds(b0, _W), :]))
                h2 = _relu(_mm(h, gm[...])
                           + _mm(oh, tc2[pl.ds(b0, _W), :]))
                acc[pl.ds(b0, _W), :] += _mm(oht, h2)

            @pl.when(jnp.logical_not(narrow))
            def _slow():
                oht = (br[0] == if_s).astype(jnp.float32)
                oh = oht.T
                h = _relu(_mm(x[...], a[...]) + _mm(oh, tc1[:_B, :]))
                h2 = _relu(_mm(h, gm[...]) + _mm(oh, tc2[:_B, :]))
                acc[:_B, :] += _mm(oht, h2)

        @pl.when(step == grid - 1)
        def _fini():
            u2s = []
            for cn, w2, b2, va, vu, d1, v2, d2, un, acc in (
                    (cn1, W21, b21, Va1, Vu1, d11, V21, d21, un1, a1),
                    (cn2, W22, b22, Va2, Vu2, d12, V22, d22, un2, a2)):
                agg2 = _mm(acc[:_B, :], w2[...]) + cn[...] * b2[...]
                zz = _relu(_mm(agg2, va[...]) + _mm(un[...], vu[...])
                           + d1[...])
                u2s.append(_mm(zz, v2[...]) + d2[...])
            zf = _relu(_mm(u2s[0], Wfa[...]) + _mm(u2s[1], Wfb[...])
                       + bf1[...])
            out[1, :, :] = _mm(zf, Wf2[...]) + bf2[...]
    return body


def kernel(x1, u1, x2, u2, batch1, batch2, params):
    n = x1.shape[0]
    R = 4000 if n % 4000 == 0 else 8
    assert n % R == 0
    grid = n // R

    (W11, b11), (W21, b21) = params['gnn1_node']
    (W12, b12), (W22, b22) = params['gnn2_node']
    (Vg11, d11), (Vg21, d21) = params['gnn1_glob']
    (Vg12, d12), (Vg22, d22) = params['gnn2_glob']
    (Wf1, bf1), (Wf2, bf2) = params['final']

    A1, Bu1 = W11[:_F], W11[_F:]
    A2, Bu2 = W12[:_F], W12[_F:]
    Va1, Vu1 = Vg11[:_F], Vg11[_F:]
    Va2, Vu2 = Vg12[:_F], Vg12[_F:]
    Wfa, Wfb = Wf1[:_F], Wf1[_F:]

    row = lambda v: v.reshape(1, -1)
    br1 = batch1.reshape(grid, 1, R)
    br2 = batch2.reshape(grid, 1, R)

    sm_spec = pl.BlockSpec(memory_space=pltpu.SMEM)
    br_spec = pl.BlockSpec((1, 1, R), lambda i: (i, 0, 0))
    x_spec = pl.BlockSpec((R, _F), lambda i: (i, 0))
    full = lambda arr: pl.BlockSpec(arr.shape, lambda i: (0,) * arr.ndim)
    acc_spec = pl.BlockSpec((_B, _F), lambda i: (0, 0))
    cnt_spec = pl.BlockSpec((_B, 1), lambda i: (0, 0))
    f32 = jnp.float32
    vmem = lambda shape: pltpu.VMEM(shape, f32)

    cnt, base_p, mx_p = _make_sc_meta(n, R, grid)(batch1, batch2)
    base = base_p[:, :grid]
    mx = mx_p[:, :grid]
    cn1 = cnt[0].reshape(_B, 1)
    cn2 = cnt[1].reshape(_B, 1)

    p1_weights = (A1, Bu1, row(b11), A2, Bu2, row(b12))
    s11, s12 = pl.pallas_call(
        _make_pass1(R, grid),
        grid=(grid,),
        in_specs=[sm_spec, sm_spec, br_spec, br_spec,
                  x_spec, x_spec, full(u1), full(u2)]
                 + [full(w) for w in p1_weights],
        out_specs=[acc_spec, acc_spec],
        out_shape=[jax.ShapeDtypeStruct((_B, _F), f32),
                   jax.ShapeDtypeStruct((_B, _F), f32)],
        scratch_shapes=[vmem((_B + _W, _F)), vmem((_B + _W, _F)),
                        vmem((_B + _W, _F)), vmem((_B + _W, _F))],
        compiler_params=pltpu.CompilerParams(
            dimension_semantics=("arbitrary",)),
    )(base, mx, br1, br2, x1, x2, u1, u2, *p1_weights)

    p2_weights = (A1, Bu1, row(b11), W21, row(b21), Va1, Vu1, row(d11),
                  Vg21, row(d21),
                  A2, Bu2, row(b12), W22, row(b22), Va2, Vu2, row(d12),
                  Vg22, row(d22),
                  Wfa, Wfb, row(bf1), Wf2, row(bf2))
    out = pl.pallas_call(
        _make_pass2(R, grid),
        grid=(grid,),
        in_specs=[sm_spec, sm_spec, br_spec, br_spec,
                  x_spec, x_spec, full(u1), full(u2), acc_spec, acc_spec,
                  cnt_spec, cnt_spec] + [full(w) for w in p2_weights],
        out_specs=pl.BlockSpec((2, _B, 2), lambda i: (0, 0, 0)),
        out_shape=jax.ShapeDtypeStruct((2, _B, 2), f32),
        scratch_shapes=[vmem((_B + _W, _F)), vmem((_B + _W, _F)),
                        vmem((_F, _F)), vmem((_B, _F)),
                        vmem((_B + _W, _F)), vmem((_B + _W, _F)),
                        vmem((_B + _W, _F)), vmem((_F, _F)),
                        vmem((_B, _F)), vmem((_B + _W, _F))],
        compiler_params=pltpu.CompilerParams(
            dimension_semantics=("arbitrary",)),
    )(base, mx, br1, br2, x1, x2, u1, u2, s11, s12, cn1, cn2,
      *p2_weights)
    return out
